# baseline (speedup 1.0000x reference)
import functools

import jax
import jax.numpy as jnp
from jax import lax
from jax.experimental import pallas as pl
from jax.experimental.pallas import tpu as pltpu
from jax.experimental.pallas import tpu_sc as plsc

B = 1024
S = 200
D = 64
N = B * S
NW = 32
PER_W = N // NW
CHUNK = 128
N_CHUNKS = PER_W // CHUNK
LANES = 16
DBLK = D // LANES
NBUF = 3


def _sc_body(seq_hbm, pos_hbm, pap_hbm, tok_tab, pos_tab, pap_tab,
             w_hbm, b_hbm, out_hbm,
             idx_tok, idx_pos, idx_pap, buf_tok, buf_pos, buf_pap,
             buf_out, w_v, b_v, gsem, osem):
    wid = lax.axis_index("s") * 2 + lax.axis_index("c")

    pltpu.sync_copy(seq_hbm.at[wid], idx_tok)
    pltpu.sync_copy(pos_hbm.at[wid], idx_pos)
    pltpu.sync_copy(pap_hbm.at[wid], idx_pap)
    pltpu.sync_copy(w_hbm, w_v)
    pltpu.sync_copy(b_hbm, b_v)

    w_tok = [w_v[0, pl.ds(j * LANES, LANES)] for j in range(DBLK)]
    w_pap = [w_v[1, pl.ds(j * LANES, LANES)] for j in range(DBLK)]
    w_pos = [w_v[2, pl.ds(j * LANES, LANES)] for j in range(DBLK)]
    bias = [b_v[pl.ds(j * LANES, LANES)] for j in range(DBLK)]

    def fire(c, slot):
        sem = gsem.at[slot]
        pltpu.async_copy(tok_tab.at[idx_tok.at[c]], buf_tok.at[slot], sem)
        pltpu.async_copy(pap_tab.at[idx_pap.at[c]], buf_pap.at[slot], sem)
        pltpu.async_copy(pos_tab.at[idx_pos.at[c]], buf_pos.at[slot], sem)

    def drain_gathers(slot):
        sem = gsem.at[slot]
        pltpu.make_async_copy(tok_tab.at[idx_tok.at[0]], buf_tok.at[slot],
                              sem).wait()
        pltpu.make_async_copy(pap_tab.at[idx_pap.at[0]], buf_pap.at[slot],
                              sem).wait()
        pltpu.make_async_copy(pos_tab.at[idx_pos.at[0]], buf_pos.at[slot],
                              sem).wait()

    for p in range(NBUF - 1):
        fire(p, p)

    def chunk_body(c, carry):
        slot = c % NBUF
        drain_gathers(slot)

        @pl.when(c + NBUF - 1 < N_CHUNKS)
        def _():
            fire(c + NBUF - 1, (c + NBUF - 1) % NBUF)

        tok, pap, pos, out = (buf_tok.at[slot], buf_pap.at[slot],
                              buf_pos.at[slot], buf_out.at[slot])
        dst = out_hbm.at[pl.ds(wid * PER_W + c * CHUNK, CHUNK), :]

        @pl.when(c >= NBUF)
        def _():
            pltpu.make_async_copy(out, dst, osem).wait()

        def row_body(r, carry2):
            for j in range(DBLK):
                ds = pl.ds(j * LANES, LANES)
                acc = tok[r, ds] * w_tok[j]
                acc += pap[r, ds] * w_pap[j]
                acc += pos[r, ds] * w_pos[j]
                out[r, ds] = acc + bias[j]
            return carry2

        lax.fori_loop(0, CHUNK, row_body, 0, unroll=4)

        pltpu.async_copy(out, dst, osem)
        return carry

    lax.fori_loop(0, N_CHUNKS, chunk_body, 0)

    for t in range(NBUF):
        c = N_CHUNKS - NBUF + t
        pltpu.make_async_copy(
            buf_out.at[c % NBUF],
            out_hbm.at[pl.ds(wid * PER_W + c * CHUNK, CHUNK), :],
            osem).wait()


def kernel(sequence, position_ids, paper_ids, token_table, position_table,
           paper_table, embedding_weights, embedding_bias):
    seq3d = sequence.reshape(NW, N_CHUNKS, CHUNK).astype(jnp.int32)
    pos3d = position_ids.reshape(NW, N_CHUNKS, CHUNK).astype(jnp.int32)
    pap3d = paper_ids.reshape(NW, N_CHUNKS, CHUNK).astype(jnp.int32)

    mesh = plsc.VectorSubcoreMesh(core_axis_name="c", subcore_axis_name="s")
    run = functools.partial(
        pl.kernel,
        mesh=mesh,
        compiler_params=pltpu.CompilerParams(use_tc_tiling_on_sc=False),
        out_type=jax.ShapeDtypeStruct((N, D), jnp.float32),
        scratch_types=[
            pltpu.VMEM((N_CHUNKS, CHUNK), jnp.int32),
            pltpu.VMEM((N_CHUNKS, CHUNK), jnp.int32),
            pltpu.VMEM((N_CHUNKS, CHUNK), jnp.int32),
            pltpu.VMEM((NBUF, CHUNK, D), jnp.float32),
            pltpu.VMEM((NBUF, CHUNK, D), jnp.float32),
            pltpu.VMEM((NBUF, CHUNK, D), jnp.float32),
            pltpu.VMEM((NBUF, CHUNK, D), jnp.float32),
            pltpu.VMEM((3, D), jnp.float32),
            pltpu.VMEM((D,), jnp.float32),
            pltpu.SemaphoreType.DMA((NBUF,)),
            pltpu.SemaphoreType.DMA,
        ],
    )(_sc_body)
    out = run(seq3d, pos3d, pap3d, token_table, position_table, paper_table,
              embedding_weights, embedding_bias)
    return out.reshape(B, S, D)


# speedup vs baseline: 1.0183x; 1.0183x over previous
import functools

import jax
import jax.numpy as jnp
from jax import lax
from jax.experimental import pallas as pl
from jax.experimental.pallas import tpu as pltpu
from jax.experimental.pallas import tpu_sc as plsc

B = 1024
S = 200
D = 64
N = B * S
NW = 32
PER_W = N // NW
CHUNK = 128
N_CHUNKS = PER_W // CHUNK
LANES = 16
DBLK = D // LANES
NBUF = 2


def _sc_body(seq_hbm, pos_hbm, pap_hbm, tok_tab, pos_tab, pap_tab,
             w_hbm, b_hbm, out_hbm,
             idx_tok, idx_pos, idx_pap, buf_tok, buf_pos, buf_pap,
             buf_out, w_v, b_v, gsem, osem):
    wid = lax.axis_index("s") * 2 + lax.axis_index("c")

    pltpu.sync_copy(seq_hbm.at[wid], idx_tok)
    pltpu.sync_copy(pos_hbm.at[wid], idx_pos)
    pltpu.sync_copy(pap_hbm.at[wid], idx_pap)
    pltpu.sync_copy(w_hbm, w_v)
    pltpu.sync_copy(b_hbm, b_v)

    w_tok = [w_v[0, pl.ds(j * LANES, LANES)] for j in range(DBLK)]
    w_pap = [w_v[1, pl.ds(j * LANES, LANES)] for j in range(DBLK)]
    w_pos = [w_v[2, pl.ds(j * LANES, LANES)] for j in range(DBLK)]
    bias = [b_v[pl.ds(j * LANES, LANES)] for j in range(DBLK)]

    def fire(c, slot):
        pltpu.async_copy(tok_tab.at[idx_tok.at[c]], buf_tok.at[slot], gsem)
        pltpu.async_copy(pap_tab.at[idx_pap.at[c]], buf_pap.at[slot], gsem)
        pltpu.async_copy(pos_tab.at[idx_pos.at[c]], buf_pos.at[slot], gsem)

    def drain_gathers(slot):
        pltpu.make_async_copy(tok_tab.at[idx_tok.at[0]], buf_tok.at[slot],
                              gsem).wait()
        pltpu.make_async_copy(pap_tab.at[idx_pap.at[0]], buf_pap.at[slot],
                              gsem).wait()
        pltpu.make_async_copy(pos_tab.at[idx_pos.at[0]], buf_pos.at[slot],
                              gsem).wait()

    fire(0, 0)

    def chunk_body(c, carry):
        slot = c % NBUF
        drain_gathers(slot)

        @pl.when(c + 1 < N_CHUNKS)
        def _():
            fire(c + 1, (c + 1) % NBUF)

        tok, pap, pos, out = (buf_tok.at[slot], buf_pap.at[slot],
                              buf_pos.at[slot], buf_out.at[slot])
        dst = out_hbm.at[pl.ds(wid * PER_W + c * CHUNK, CHUNK), :]

        @pl.when(c >= NBUF)
        def _():
            pltpu.make_async_copy(out, dst, osem).wait()

        def row_body(r, carry2):
            for j in range(DBLK):
                ds = pl.ds(j * LANES, LANES)
                acc = tok[r, ds] * w_tok[j]
                acc += pap[r, ds] * w_pap[j]
                acc += pos[r, ds] * w_pos[j]
                out[r, ds] = acc + bias[j]
            return carry2

        lax.fori_loop(0, CHUNK, row_body, 0, unroll=4)

        pltpu.async_copy(out, dst, osem)
        return carry

    lax.fori_loop(0, N_CHUNKS, chunk_body, 0)

    for t in range(NBUF):
        c = N_CHUNKS - NBUF + t
        pltpu.make_async_copy(
            buf_out.at[c % NBUF],
            out_hbm.at[pl.ds(wid * PER_W + c * CHUNK, CHUNK), :],
            osem).wait()


def kernel(sequence, position_ids, paper_ids, token_table, position_table,
           paper_table, embedding_weights, embedding_bias):
    seq3d = sequence.reshape(NW, N_CHUNKS, CHUNK).astype(jnp.int32)
    pos3d = position_ids.reshape(NW, N_CHUNKS, CHUNK).astype(jnp.int32)
    pap3d = paper_ids.reshape(NW, N_CHUNKS, CHUNK).astype(jnp.int32)

    mesh = plsc.VectorSubcoreMesh(core_axis_name="c", subcore_axis_name="s")
    run = functools.partial(
        pl.kernel,
        mesh=mesh,
        compiler_params=pltpu.CompilerParams(use_tc_tiling_on_sc=False),
        out_type=jax.ShapeDtypeStruct((N, D), jnp.float32),
        scratch_types=[
            pltpu.VMEM((N_CHUNKS, CHUNK), jnp.int32),
            pltpu.VMEM((N_CHUNKS, CHUNK), jnp.int32),
            pltpu.VMEM((N_CHUNKS, CHUNK), jnp.int32),
            pltpu.VMEM((NBUF, CHUNK, D), jnp.float32),
            pltpu.VMEM((NBUF, CHUNK, D), jnp.float32),
            pltpu.VMEM((NBUF, CHUNK, D), jnp.float32),
            pltpu.VMEM((NBUF, CHUNK, D), jnp.float32),
            pltpu.VMEM((3, D), jnp.float32),
            pltpu.VMEM((D,), jnp.float32),
            pltpu.SemaphoreType.DMA,
            pltpu.SemaphoreType.DMA,
        ],
    )(_sc_body)
    out = run(seq3d, pos3d, pap3d, token_table, position_table, paper_table,
              embedding_weights, embedding_bias)
    return out.reshape(B, S, D)
